# Initial kernel scaffold; baseline (speedup 1.0000x reference)
import jax
import jax.numpy as jnp
from jax.experimental import pallas as pl


def kernel(input_embedding, centroids, codebooks, db_assign, db_codes):
    raise NotImplementedError("write your pallas kernel here")


# speedup vs baseline: 1.4252x; 1.4252x over previous
import functools

import jax
import jax.numpy as jnp
from jax import lax
from jax.experimental import pallas as pl

_NPROBE = 32
_TOPK = 100


def _adc_block_kernel(C, M, KK, t0_ref, t1_ref, t2_ref, keys_ref, term_ref,
                      tau_ref, out_ref):
    nb = keys_ref.shape[1]
    keys = keys_ref[...]
    dn = (((1,), (0,)), ((), ()))

    def exact_gather(lo, hi, oh):
        p0 = lax.dot_general(t0_ref[:, lo:hi], oh, dn,
                             preferred_element_type=jnp.float32)
        p1 = lax.dot_general(t1_ref[:, lo:hi], oh, dn,
                             preferred_element_type=jnp.float32)
        p2 = lax.dot_general(t2_ref[:, lo:hi], oh, dn,
                             preferred_element_type=jnp.float32)
        return (p0 + p1) + p2

    iota_c = lax.broadcasted_iota(jnp.int32, (C, nb), 0)
    ohc = (iota_c == keys[0:1, :]).astype(jnp.bfloat16)
    cg = exact_gather(0, C, ohc)
    d = cg + term_ref[0]
    iota_k = lax.broadcasted_iota(jnp.int32, (KK, nb), 0)
    for m in range(M):
        cm = keys[m + 1:m + 2, :] - (C + KK * m)
        ohm = (iota_k == cm).astype(jnp.bfloat16)
        d = d - 2.0 * exact_gather(C + KK * m, C + KK * (m + 1), ohm)
    out_ref[...] = jnp.where(cg <= tau_ref[...], d, jnp.float32(1e30))


def _split3(t):
    mask = jnp.uint32(0xFFFF0000)
    t0f = lax.bitcast_convert_type(
        lax.bitcast_convert_type(t, jnp.uint32) & mask, jnp.float32)
    r1 = t - t0f
    t1f = lax.bitcast_convert_type(
        lax.bitcast_convert_type(r1, jnp.uint32) & mask, jnp.float32)
    r2 = r1 - t1f
    return (t0f.astype(jnp.bfloat16), t1f.astype(jnp.bfloat16),
            r2.astype(jnp.bfloat16))


def _ivfpq_search(q, centroids, codebooks, db_assign, db_codes,
                  nprobe, topk, qb, nb):
    Q, D = q.shape
    C = centroids.shape[0]
    M, KK, DS = codebooks.shape
    N = db_assign.shape[0]

    qn = jnp.sum(q * q, axis=1, keepdims=True)
    cn = jnp.sum(centroids * centroids, axis=1)[None, :]
    coarse = qn + cn - 2.0 * (q @ centroids.T)
    tau = -lax.top_k(-coarse, nprobe)[0][:, -1:]

    qs = q.reshape(Q, M, DS)
    cs = centroids.reshape(C, M, DS)
    qdot = jnp.einsum('qmd,mkd->qmk', qs, codebooks)
    cdot = jnp.einsum('cmd,mkd->cmk', cs, codebooks)
    rnorm = jnp.sum(codebooks * codebooks, axis=2)

    term = jnp.zeros((N,), dtype=jnp.float32)
    for m in range(M):
        cd = cdot[db_assign, m, db_codes[:, m]]
        term = term + 2.0 * cd + rnorm[m, db_codes[:, m]]

    table = jnp.concatenate([coarse, qdot.reshape(Q, M * KK)], axis=1)
    t0, t1, t2 = _split3(table)

    n_blocks = -(-N // nb)
    n_pad = n_blocks * nb
    pad = n_pad - N
    a_p = jnp.concatenate([db_assign, jnp.zeros((pad,), jnp.int32)])
    codes_p = jnp.concatenate([db_codes, jnp.zeros((pad, M), jnp.int32)], axis=0)
    term_p = jnp.concatenate([term, jnp.full((pad,), jnp.inf, jnp.float32)])
    offs = C + KK * jnp.arange(M, dtype=jnp.int32)
    keys = jnp.concatenate([a_p[None, :], (codes_p + offs[None, :]).T], axis=0)
    keys = keys.astype(jnp.int32)
    term3 = term_p.reshape(n_blocks, 1, nb)

    q_blocks = Q // qb
    dist = pl.pallas_call(
        functools.partial(_adc_block_kernel, C, M, KK),
        grid=(q_blocks, n_blocks),
        in_specs=[
            pl.BlockSpec((qb, C + M * KK), lambda i, j: (i, 0)),
            pl.BlockSpec((qb, C + M * KK), lambda i, j: (i, 0)),
            pl.BlockSpec((qb, C + M * KK), lambda i, j: (i, 0)),
            pl.BlockSpec((1 + M, nb), lambda i, j: (0, j)),
            pl.BlockSpec((1, 1, nb), lambda i, j: (j, 0, 0)),
            pl.BlockSpec((qb, 1), lambda i, j: (i, 0)),
        ],
        out_specs=pl.BlockSpec((qb, nb), lambda i, j: (i, j)),
        out_shape=jax.ShapeDtypeStruct((Q, n_pad), jnp.float32),
    )(t0, t1, t2, keys, term3, tau)

    _, indices = lax.top_k(-dist, topk)
    return indices


def kernel(input_embedding, centroids, codebooks, db_assign, db_codes):
    return _ivfpq_search(input_embedding, centroids, codebooks,
                         db_assign, db_codes,
                         nprobe=_NPROBE, topk=_TOPK, qb=512, nb=512)


# speedup vs baseline: 2.0050x; 1.4068x over previous
import functools

import jax
import jax.numpy as jnp
from jax import lax
from jax.experimental import pallas as pl

_NPROBE = 32
_TOPK = 100


def _adc_block_kernel(C, M, KK, t0_ref, t1_ref, t2_ref, keys_ref, term_ref,
                      tau_ref, out_ref):
    nb = keys_ref.shape[1]
    keys = keys_ref[...]
    dn = (((1,), (0,)), ((), ()))

    def exact_gather(lo, hi, oh):
        p0 = lax.dot_general(t0_ref[:, lo:hi], oh, dn,
                             preferred_element_type=jnp.float32)
        p1 = lax.dot_general(t1_ref[:, lo:hi], oh, dn,
                             preferred_element_type=jnp.float32)
        p2 = lax.dot_general(t2_ref[:, lo:hi], oh, dn,
                             preferred_element_type=jnp.float32)
        return (p0 + p1) + p2

    iota_c = lax.broadcasted_iota(jnp.int32, (C, nb), 0)
    ohc = (iota_c == keys[0:1, :]).astype(jnp.bfloat16)
    cg = exact_gather(0, C, ohc)
    d = cg + term_ref[0]
    iota_k = lax.broadcasted_iota(jnp.int32, (KK, nb), 0)
    for m in range(M):
        cm = keys[m + 1:m + 2, :] - (C + KK * m)
        ohm = (iota_k == cm).astype(jnp.bfloat16)
        d = d - 2.0 * exact_gather(C + KK * m, C + KK * (m + 1), ohm)
    out_ref[...] = jnp.where(cg <= tau_ref[...], d, jnp.float32(1e30))


def _term_block_kernel(C, M, KK, cd0_ref, cd1_ref, cd2_ref, rn_ref, keys_ref,
                       out_ref):
    nb = keys_ref.shape[1]
    keys = keys_ref[...]
    dn = (((1,), (0,)), ((), ()))
    iota_c = lax.broadcasted_iota(jnp.int32, (C, nb), 0)
    ohc = (iota_c == keys[0:1, :]).astype(jnp.bfloat16)
    iota_k = lax.broadcasted_iota(jnp.int32, (KK, nb), 0)
    pieces = []
    for m in range(M):
        cm = keys[m + 1:m + 2, :] - (C + KK * m)
        pieces.append((iota_k == cm).astype(jnp.float32))
    ohp = jnp.concatenate(pieces, axis=0)
    r0 = lax.dot_general(cd0_ref[...], ohc, dn, preferred_element_type=jnp.float32)
    r1 = lax.dot_general(cd1_ref[...], ohc, dn, preferred_element_type=jnp.float32)
    r2 = lax.dot_general(cd2_ref[...], ohc, dn, preferred_element_type=jnp.float32)
    rows = (r0 + r1) + r2
    cd_pick = jnp.sum((rows * ohp).reshape(M, KK, nb), axis=1)
    rn_pick = jnp.sum((rn_ref[...] * ohp).reshape(M, KK, nb), axis=1)
    t = jnp.zeros((1, nb), jnp.float32)
    for m in range(M):
        t = t + 2.0 * cd_pick[m:m + 1, :] + rn_pick[m:m + 1, :]
    out_ref[...] = t.reshape(1, 1, nb)


def _split3(t):
    mask = jnp.uint32(0xFFFF0000)
    t0f = lax.bitcast_convert_type(
        lax.bitcast_convert_type(t, jnp.uint32) & mask, jnp.float32)
    r1 = t - t0f
    t1f = lax.bitcast_convert_type(
        lax.bitcast_convert_type(r1, jnp.uint32) & mask, jnp.float32)
    r2 = r1 - t1f
    return (t0f.astype(jnp.bfloat16), t1f.astype(jnp.bfloat16),
            r2.astype(jnp.bfloat16))


def _ivfpq_search(q, centroids, codebooks, db_assign, db_codes,
                  nprobe, topk, qb, nb):
    Q, D = q.shape
    C = centroids.shape[0]
    M, KK, DS = codebooks.shape
    N = db_assign.shape[0]

    qn = jnp.sum(q * q, axis=1, keepdims=True)
    cn = jnp.sum(centroids * centroids, axis=1)[None, :]
    coarse = qn + cn - 2.0 * (q @ centroids.T)
    tau = -lax.top_k(-coarse, nprobe)[0][:, -1:]

    qs = q.reshape(Q, M, DS)
    cs = centroids.reshape(C, M, DS)
    qdot = jnp.einsum('qmd,mkd->qmk', qs, codebooks)
    cdot = jnp.einsum('cmd,mkd->cmk', cs, codebooks)
    rnorm = jnp.sum(codebooks * codebooks, axis=2)

    table = jnp.concatenate([coarse, qdot.reshape(Q, M * KK)], axis=1)
    t0, t1, t2 = _split3(table)

    n_blocks = -(-N // nb)
    n_pad = n_blocks * nb
    pad = n_pad - N
    a_p = jnp.concatenate([db_assign, jnp.zeros((pad,), jnp.int32)])
    codes_p = jnp.concatenate([db_codes, jnp.zeros((pad, M), jnp.int32)], axis=0)
    offs = C + KK * jnp.arange(M, dtype=jnp.int32)
    keys = jnp.concatenate([a_p[None, :], (codes_p + offs[None, :]).T], axis=0)
    keys = keys.astype(jnp.int32)

    cdT = jnp.transpose(cdot.reshape(C, M * KK))
    c0, c1, c2 = _split3(cdT)
    rncol = rnorm.reshape(M * KK, 1)
    term3 = pl.pallas_call(
        functools.partial(_term_block_kernel, C, M, KK),
        grid=(n_blocks,),
        in_specs=[
            pl.BlockSpec((M * KK, C), lambda j: (0, 0)),
            pl.BlockSpec((M * KK, C), lambda j: (0, 0)),
            pl.BlockSpec((M * KK, C), lambda j: (0, 0)),
            pl.BlockSpec((M * KK, 1), lambda j: (0, 0)),
            pl.BlockSpec((1 + M, nb), lambda j: (0, j)),
        ],
        out_specs=pl.BlockSpec((1, 1, nb), lambda j: (j, 0, 0)),
        out_shape=jax.ShapeDtypeStruct((n_blocks, 1, nb), jnp.float32),
    )(c0, c1, c2, rncol, keys)
    if pad:
        tflat = term3.reshape(n_pad)
        tflat = jnp.where(jnp.arange(n_pad) >= N, jnp.inf, tflat)
        term3 = tflat.reshape(n_blocks, 1, nb)

    q_blocks = Q // qb
    dist = pl.pallas_call(
        functools.partial(_adc_block_kernel, C, M, KK),
        grid=(q_blocks, n_blocks),
        in_specs=[
            pl.BlockSpec((qb, C + M * KK), lambda i, j: (i, 0)),
            pl.BlockSpec((qb, C + M * KK), lambda i, j: (i, 0)),
            pl.BlockSpec((qb, C + M * KK), lambda i, j: (i, 0)),
            pl.BlockSpec((1 + M, nb), lambda i, j: (0, j)),
            pl.BlockSpec((1, 1, nb), lambda i, j: (j, 0, 0)),
            pl.BlockSpec((qb, 1), lambda i, j: (i, 0)),
        ],
        out_specs=pl.BlockSpec((qb, nb), lambda i, j: (i, j)),
        out_shape=jax.ShapeDtypeStruct((Q, n_pad), jnp.float32),
    )(t0, t1, t2, keys, term3, tau)

    _, indices = lax.top_k(-dist, topk)
    return indices


def kernel(input_embedding, centroids, codebooks, db_assign, db_codes):
    return _ivfpq_search(input_embedding, centroids, codebooks,
                         db_assign, db_codes,
                         nprobe=_NPROBE, topk=_TOPK, qb=512, nb=512)


# speedup vs baseline: 2.0075x; 1.0012x over previous
import functools

import jax
import jax.numpy as jnp
from jax import lax
from jax.experimental import pallas as pl

_NPROBE = 32
_TOPK = 100


def _adc_block_kernel(C, M, KK, t0_ref, t1_ref, t2_ref, keys_ref, term_ref,
                      tau_ref, out_ref):
    nb = keys_ref.shape[1]
    keys = keys_ref[...]
    dn = (((1,), (0,)), ((), ()))

    def exact_gather(lo, hi, oh):
        p0 = lax.dot_general(t0_ref[:, lo:hi], oh, dn,
                             preferred_element_type=jnp.float32)
        p1 = lax.dot_general(t1_ref[:, lo:hi], oh, dn,
                             preferred_element_type=jnp.float32)
        p2 = lax.dot_general(t2_ref[:, lo:hi], oh, dn,
                             preferred_element_type=jnp.float32)
        return (p0 + p1) + p2

    iota_c = lax.broadcasted_iota(jnp.int32, (C, nb), 0)
    ohc = (iota_c == keys[0:1, :]).astype(jnp.bfloat16)
    cg = exact_gather(0, C, ohc)
    d = cg + term_ref[0]
    iota_k = lax.broadcasted_iota(jnp.int32, (KK, nb), 0)
    for m in range(M):
        cm = keys[m + 1:m + 2, :] - (C + KK * m)
        ohm = (iota_k == cm).astype(jnp.bfloat16)
        d = d - 2.0 * exact_gather(C + KK * m, C + KK * (m + 1), ohm)
    out_ref[...] = jnp.where(cg <= tau_ref[...], -d, jnp.float32(-1e30))


def _term_block_kernel(C, M, KK, cd0_ref, cd1_ref, cd2_ref, rn_ref, keys_ref,
                       out_ref):
    nb = keys_ref.shape[1]
    keys = keys_ref[...]
    dn = (((1,), (0,)), ((), ()))
    iota_c = lax.broadcasted_iota(jnp.int32, (C, nb), 0)
    ohc = (iota_c == keys[0:1, :]).astype(jnp.bfloat16)
    iota_k = lax.broadcasted_iota(jnp.int32, (KK, nb), 0)
    pieces = []
    for m in range(M):
        cm = keys[m + 1:m + 2, :] - (C + KK * m)
        pieces.append((iota_k == cm).astype(jnp.float32))
    ohp = jnp.concatenate(pieces, axis=0)
    r0 = lax.dot_general(cd0_ref[...], ohc, dn, preferred_element_type=jnp.float32)
    r1 = lax.dot_general(cd1_ref[...], ohc, dn, preferred_element_type=jnp.float32)
    r2 = lax.dot_general(cd2_ref[...], ohc, dn, preferred_element_type=jnp.float32)
    rows = (r0 + r1) + r2
    cd_pick = jnp.sum((rows * ohp).reshape(M, KK, nb), axis=1)
    rn_pick = jnp.sum((rn_ref[...] * ohp).reshape(M, KK, nb), axis=1)
    t = jnp.zeros((1, nb), jnp.float32)
    for m in range(M):
        t = t + 2.0 * cd_pick[m:m + 1, :] + rn_pick[m:m + 1, :]
    out_ref[...] = t.reshape(1, 1, nb)


def _split3(t):
    mask = jnp.uint32(0xFFFF0000)
    t0f = lax.bitcast_convert_type(
        lax.bitcast_convert_type(t, jnp.uint32) & mask, jnp.float32)
    r1 = t - t0f
    t1f = lax.bitcast_convert_type(
        lax.bitcast_convert_type(r1, jnp.uint32) & mask, jnp.float32)
    r2 = r1 - t1f
    return (t0f.astype(jnp.bfloat16), t1f.astype(jnp.bfloat16),
            r2.astype(jnp.bfloat16))


def _ivfpq_search(q, centroids, codebooks, db_assign, db_codes,
                  nprobe, topk, qb, nb):
    Q, D = q.shape
    C = centroids.shape[0]
    M, KK, DS = codebooks.shape
    N = db_assign.shape[0]

    qn = jnp.sum(q * q, axis=1, keepdims=True)
    cn = jnp.sum(centroids * centroids, axis=1)[None, :]
    coarse = qn + cn - 2.0 * (q @ centroids.T)
    tau = -lax.top_k(-coarse, nprobe)[0][:, -1:]

    qs = q.reshape(Q, M, DS)
    cs = centroids.reshape(C, M, DS)
    qdot = jnp.einsum('qmd,mkd->qmk', qs, codebooks)
    cdot = jnp.einsum('cmd,mkd->cmk', cs, codebooks)
    rnorm = jnp.sum(codebooks * codebooks, axis=2)

    table = jnp.concatenate([coarse, qdot.reshape(Q, M * KK)], axis=1)
    t0, t1, t2 = _split3(table)

    n_blocks = -(-N // nb)
    n_pad = n_blocks * nb
    pad = n_pad - N
    a_p = jnp.concatenate([db_assign, jnp.zeros((pad,), jnp.int32)])
    codes_p = jnp.concatenate([db_codes, jnp.zeros((pad, M), jnp.int32)], axis=0)
    offs = C + KK * jnp.arange(M, dtype=jnp.int32)
    keys = jnp.concatenate([a_p[None, :], (codes_p + offs[None, :]).T], axis=0)
    keys = keys.astype(jnp.int32)

    cdT = jnp.transpose(cdot.reshape(C, M * KK))
    c0, c1, c2 = _split3(cdT)
    rncol = rnorm.reshape(M * KK, 1)
    term3 = pl.pallas_call(
        functools.partial(_term_block_kernel, C, M, KK),
        grid=(n_blocks,),
        in_specs=[
            pl.BlockSpec((M * KK, C), lambda j: (0, 0)),
            pl.BlockSpec((M * KK, C), lambda j: (0, 0)),
            pl.BlockSpec((M * KK, C), lambda j: (0, 0)),
            pl.BlockSpec((M * KK, 1), lambda j: (0, 0)),
            pl.BlockSpec((1 + M, nb), lambda j: (0, j)),
        ],
        out_specs=pl.BlockSpec((1, 1, nb), lambda j: (j, 0, 0)),
        out_shape=jax.ShapeDtypeStruct((n_blocks, 1, nb), jnp.float32),
    )(c0, c1, c2, rncol, keys)
    if pad:
        tflat = term3.reshape(n_pad)
        tflat = jnp.where(jnp.arange(n_pad) >= N, jnp.inf, tflat)
        term3 = tflat.reshape(n_blocks, 1, nb)

    q_blocks = Q // qb
    dist = pl.pallas_call(
        functools.partial(_adc_block_kernel, C, M, KK),
        grid=(q_blocks, n_blocks),
        in_specs=[
            pl.BlockSpec((qb, C + M * KK), lambda i, j: (i, 0)),
            pl.BlockSpec((qb, C + M * KK), lambda i, j: (i, 0)),
            pl.BlockSpec((qb, C + M * KK), lambda i, j: (i, 0)),
            pl.BlockSpec((1 + M, nb), lambda i, j: (0, j)),
            pl.BlockSpec((1, 1, nb), lambda i, j: (j, 0, 0)),
            pl.BlockSpec((qb, 1), lambda i, j: (i, 0)),
        ],
        out_specs=pl.BlockSpec((qb, nb), lambda i, j: (i, j)),
        out_shape=jax.ShapeDtypeStruct((Q, n_pad), jnp.float32),
    )(t0, t1, t2, keys, term3, tau)

    _, indices = lax.top_k(dist, topk)
    return indices


def kernel(input_embedding, centroids, codebooks, db_assign, db_codes):
    return _ivfpq_search(input_embedding, centroids, codebooks,
                         db_assign, db_codes,
                         nprobe=_NPROBE, topk=_TOPK, qb=512, nb=512)


# speedup vs baseline: 7.7749x; 3.8730x over previous
import functools

import jax
import jax.numpy as jnp
from jax import lax
from jax.experimental import pallas as pl

_NPROBE = 32
_TOPK = 100


def _adc_block_kernel(C, M, KK, t0_ref, t1_ref, t2_ref, keys_ref, term_ref,
                      tau_ref, out_ref):
    nb = keys_ref.shape[1]
    keys = keys_ref[...]
    dn = (((1,), (0,)), ((), ()))

    def exact_gather(lo, hi, oh):
        p0 = lax.dot_general(t0_ref[:, lo:hi], oh, dn,
                             preferred_element_type=jnp.float32)
        p1 = lax.dot_general(t1_ref[:, lo:hi], oh, dn,
                             preferred_element_type=jnp.float32)
        p2 = lax.dot_general(t2_ref[:, lo:hi], oh, dn,
                             preferred_element_type=jnp.float32)
        return (p0 + p1) + p2

    iota_c = lax.broadcasted_iota(jnp.int32, (C, nb), 0)
    ohc = (iota_c == keys[0:1, :]).astype(jnp.bfloat16)
    cg = exact_gather(0, C, ohc)
    d = cg + term_ref[0]
    iota_k = lax.broadcasted_iota(jnp.int32, (KK, nb), 0)
    for m in range(M):
        cm = keys[m + 1:m + 2, :] - (C + KK * m)
        ohm = (iota_k == cm).astype(jnp.bfloat16)
        d = d - 2.0 * exact_gather(C + KK * m, C + KK * (m + 1), ohm)
    out_ref[...] = jnp.where(cg <= tau_ref[...], -d, jnp.float32(-1e30))


def _term_block_kernel(C, M, KK, cd0_ref, cd1_ref, cd2_ref, rn_ref, keys_ref,
                       out_ref):
    nb = keys_ref.shape[1]
    keys = keys_ref[...]
    dn = (((1,), (0,)), ((), ()))
    iota_c = lax.broadcasted_iota(jnp.int32, (C, nb), 0)
    ohc = (iota_c == keys[0:1, :]).astype(jnp.bfloat16)
    iota_k = lax.broadcasted_iota(jnp.int32, (KK, nb), 0)
    pieces = []
    for m in range(M):
        cm = keys[m + 1:m + 2, :] - (C + KK * m)
        pieces.append((iota_k == cm).astype(jnp.float32))
    ohp = jnp.concatenate(pieces, axis=0)
    r0 = lax.dot_general(cd0_ref[...], ohc, dn, preferred_element_type=jnp.float32)
    r1 = lax.dot_general(cd1_ref[...], ohc, dn, preferred_element_type=jnp.float32)
    r2 = lax.dot_general(cd2_ref[...], ohc, dn, preferred_element_type=jnp.float32)
    rows = (r0 + r1) + r2
    cd_pick = jnp.sum((rows * ohp).reshape(M, KK, nb), axis=1)
    rn_pick = jnp.sum((rn_ref[...] * ohp).reshape(M, KK, nb), axis=1)
    t = jnp.zeros((1, nb), jnp.float32)
    for m in range(M):
        t = t + 2.0 * cd_pick[m:m + 1, :] + rn_pick[m:m + 1, :]
    out_ref[...] = t.reshape(1, 1, nb)


_NSLOT = 8


def _extract_block_kernel(n_pad, nb, v_ref, vals_ref, idx_ref, v9_ref):
    v = v_ref[...]
    qb = v.shape[0]
    li = lax.broadcasted_iota(jnp.int32, (qb, nb), 1)
    base = pl.program_id(1) * nb
    vs, ids = [], []
    for _ in range(_NSLOT):
        m = jnp.max(v, axis=1, keepdims=True)
        am = jnp.min(jnp.where(v == m, li, n_pad), axis=1, keepdims=True)
        vs.append(m)
        ids.append(am + base)
        v = jnp.where(li == am, -jnp.inf, v)
    vals_ref[...] = jnp.concatenate(vs, axis=1).reshape(1, qb, _NSLOT)
    idx_ref[...] = jnp.concatenate(ids, axis=1).reshape(1, qb, _NSLOT)
    v9_ref[...] = jnp.max(v, axis=1, keepdims=True).reshape(1, qb, 1)


def _select_topk(dist, topk, qb, nb):
    Q, n_pad = dist.shape
    n_blocks = n_pad // nb
    vals, idxs, v9 = pl.pallas_call(
        functools.partial(_extract_block_kernel, n_pad, nb),
        grid=(Q // qb, n_blocks),
        in_specs=[pl.BlockSpec((qb, nb), lambda i, j: (i, j))],
        out_specs=[
            pl.BlockSpec((1, qb, _NSLOT), lambda i, j: (j, i, 0)),
            pl.BlockSpec((1, qb, _NSLOT), lambda i, j: (j, i, 0)),
            pl.BlockSpec((1, qb, 1), lambda i, j: (j, i, 0)),
        ],
        out_shape=[
            jax.ShapeDtypeStruct((n_blocks, Q, _NSLOT), jnp.float32),
            jax.ShapeDtypeStruct((n_blocks, Q, _NSLOT), jnp.int32),
            jax.ShapeDtypeStruct((n_blocks, Q, 1), jnp.float32),
        ],
    )(dist)
    cand_v = vals.transpose(1, 0, 2).reshape(Q, n_blocks * _NSLOT)
    cand_i = idxs.transpose(1, 0, 2).reshape(Q, n_blocks * _NSLOT)
    tv, tp = lax.top_k(cand_v, topk)
    fast_idx = jnp.take_along_axis(cand_i, tp, axis=1)
    kth = tv[:, topk - 1]
    v9max = jnp.max(v9[:, :, 0], axis=0)
    ok_fast = jnp.all(v9max < kth)
    return lax.cond(ok_fast,
                    lambda _: fast_idx,
                    lambda _: lax.top_k(dist, topk)[1],
                    operand=None)


def _split3(t):
    mask = jnp.uint32(0xFFFF0000)
    t0f = lax.bitcast_convert_type(
        lax.bitcast_convert_type(t, jnp.uint32) & mask, jnp.float32)
    r1 = t - t0f
    t1f = lax.bitcast_convert_type(
        lax.bitcast_convert_type(r1, jnp.uint32) & mask, jnp.float32)
    r2 = r1 - t1f
    return (t0f.astype(jnp.bfloat16), t1f.astype(jnp.bfloat16),
            r2.astype(jnp.bfloat16))


def _ivfpq_search(q, centroids, codebooks, db_assign, db_codes,
                  nprobe, topk, qb, nb):
    Q, D = q.shape
    C = centroids.shape[0]
    M, KK, DS = codebooks.shape
    N = db_assign.shape[0]

    qn = jnp.sum(q * q, axis=1, keepdims=True)
    cn = jnp.sum(centroids * centroids, axis=1)[None, :]
    coarse = qn + cn - 2.0 * (q @ centroids.T)
    tau = -lax.top_k(-coarse, nprobe)[0][:, -1:]

    qs = q.reshape(Q, M, DS)
    cs = centroids.reshape(C, M, DS)
    qdot = jnp.einsum('qmd,mkd->qmk', qs, codebooks)
    cdot = jnp.einsum('cmd,mkd->cmk', cs, codebooks)
    rnorm = jnp.sum(codebooks * codebooks, axis=2)

    table = jnp.concatenate([coarse, qdot.reshape(Q, M * KK)], axis=1)
    t0, t1, t2 = _split3(table)

    n_blocks = -(-N // nb)
    n_pad = n_blocks * nb
    pad = n_pad - N
    a_p = jnp.concatenate([db_assign, jnp.zeros((pad,), jnp.int32)])
    codes_p = jnp.concatenate([db_codes, jnp.zeros((pad, M), jnp.int32)], axis=0)
    offs = C + KK * jnp.arange(M, dtype=jnp.int32)
    keys = jnp.concatenate([a_p[None, :], (codes_p + offs[None, :]).T], axis=0)
    keys = keys.astype(jnp.int32)

    cdT = jnp.transpose(cdot.reshape(C, M * KK))
    c0, c1, c2 = _split3(cdT)
    rncol = rnorm.reshape(M * KK, 1)
    term3 = pl.pallas_call(
        functools.partial(_term_block_kernel, C, M, KK),
        grid=(n_blocks,),
        in_specs=[
            pl.BlockSpec((M * KK, C), lambda j: (0, 0)),
            pl.BlockSpec((M * KK, C), lambda j: (0, 0)),
            pl.BlockSpec((M * KK, C), lambda j: (0, 0)),
            pl.BlockSpec((M * KK, 1), lambda j: (0, 0)),
            pl.BlockSpec((1 + M, nb), lambda j: (0, j)),
        ],
        out_specs=pl.BlockSpec((1, 1, nb), lambda j: (j, 0, 0)),
        out_shape=jax.ShapeDtypeStruct((n_blocks, 1, nb), jnp.float32),
    )(c0, c1, c2, rncol, keys)
    if pad:
        tflat = term3.reshape(n_pad)
        tflat = jnp.where(jnp.arange(n_pad) >= N, jnp.inf, tflat)
        term3 = tflat.reshape(n_blocks, 1, nb)

    q_blocks = Q // qb
    dist = pl.pallas_call(
        functools.partial(_adc_block_kernel, C, M, KK),
        grid=(q_blocks, n_blocks),
        in_specs=[
            pl.BlockSpec((qb, C + M * KK), lambda i, j: (i, 0)),
            pl.BlockSpec((qb, C + M * KK), lambda i, j: (i, 0)),
            pl.BlockSpec((qb, C + M * KK), lambda i, j: (i, 0)),
            pl.BlockSpec((1 + M, nb), lambda i, j: (0, j)),
            pl.BlockSpec((1, 1, nb), lambda i, j: (j, 0, 0)),
            pl.BlockSpec((qb, 1), lambda i, j: (i, 0)),
        ],
        out_specs=pl.BlockSpec((qb, nb), lambda i, j: (i, j)),
        out_shape=jax.ShapeDtypeStruct((Q, n_pad), jnp.float32),
    )(t0, t1, t2, keys, term3, tau)

    return _select_topk(dist, topk, qb, nb)


def kernel(input_embedding, centroids, codebooks, db_assign, db_codes):
    return _ivfpq_search(input_embedding, centroids, codebooks,
                         db_assign, db_codes,
                         nprobe=_NPROBE, topk=_TOPK, qb=512, nb=512)


# speedup vs baseline: 8.0893x; 1.0404x over previous
import functools

import jax
import jax.numpy as jnp
from jax import lax
from jax.experimental import pallas as pl

_NPROBE = 32
_TOPK = 100


def _adc_block_kernel(C, M, KK, t0_ref, t1_ref, t2_ref, keys_ref, term_ref,
                      tau_ref, out_ref):
    nb = keys_ref.shape[1]
    keys = keys_ref[...]
    dn = (((1,), (0,)), ((), ()))

    def exact_gather(lo, hi, oh):
        p0 = lax.dot_general(t0_ref[:, lo:hi], oh, dn,
                             preferred_element_type=jnp.float32)
        p1 = lax.dot_general(t1_ref[:, lo:hi], oh, dn,
                             preferred_element_type=jnp.float32)
        p2 = lax.dot_general(t2_ref[:, lo:hi], oh, dn,
                             preferred_element_type=jnp.float32)
        return (p0 + p1) + p2

    iota_c = lax.broadcasted_iota(jnp.int32, (C, nb), 0)
    ohc = (iota_c == keys[0:1, :]).astype(jnp.bfloat16)
    cg = exact_gather(0, C, ohc)
    d = cg + term_ref[0]
    iota_k = lax.broadcasted_iota(jnp.int32, (KK, nb), 0)
    for m in range(M):
        cm = keys[m + 1:m + 2, :] - (C + KK * m)
        ohm = (iota_k == cm).astype(jnp.bfloat16)
        d = d - 2.0 * exact_gather(C + KK * m, C + KK * (m + 1), ohm)
    out_ref[...] = jnp.where(cg <= tau_ref[...], -d, jnp.float32(-1e30))


def _term_block_kernel(C, M, KK, cd0_ref, cd1_ref, cd2_ref, rn_ref, keys_ref,
                       out_ref):
    nb = keys_ref.shape[1]
    keys = keys_ref[...]
    dn = (((1,), (0,)), ((), ()))
    iota_c = lax.broadcasted_iota(jnp.int32, (C, nb), 0)
    ohc = (iota_c == keys[0:1, :]).astype(jnp.bfloat16)
    iota_k = lax.broadcasted_iota(jnp.int32, (KK, nb), 0)
    pieces = []
    for m in range(M):
        cm = keys[m + 1:m + 2, :] - (C + KK * m)
        pieces.append((iota_k == cm).astype(jnp.float32))
    ohp = jnp.concatenate(pieces, axis=0)
    r0 = lax.dot_general(cd0_ref[...], ohc, dn, preferred_element_type=jnp.float32)
    r1 = lax.dot_general(cd1_ref[...], ohc, dn, preferred_element_type=jnp.float32)
    r2 = lax.dot_general(cd2_ref[...], ohc, dn, preferred_element_type=jnp.float32)
    rows = (r0 + r1) + r2
    cd_pick = jnp.sum((rows * ohp).reshape(M, KK, nb), axis=1)
    rn_pick = jnp.sum((rn_ref[...] * ohp).reshape(M, KK, nb), axis=1)
    t = jnp.zeros((1, nb), jnp.float32)
    for m in range(M):
        t = t + 2.0 * cd_pick[m:m + 1, :] + rn_pick[m:m + 1, :]
    out_ref[...] = t.reshape(1, 1, nb)


_NSLOT = 8


def _extract_block_kernel(n_pad, nb, v_ref, vals_ref, idx_ref, v9_ref):
    v = v_ref[...]
    qb = v.shape[0]
    li = lax.broadcasted_iota(jnp.int32, (qb, nb), 1)
    base = pl.program_id(1) * nb
    vs, ids = [], []
    for _ in range(_NSLOT):
        m = jnp.max(v, axis=1, keepdims=True)
        am = jnp.min(jnp.where(v == m, li, n_pad), axis=1, keepdims=True)
        vs.append(m)
        ids.append(am + base)
        v = jnp.where(li == am, -jnp.inf, v)
    vals_ref[...] = jnp.concatenate(vs, axis=1).reshape(1, qb, _NSLOT)
    idx_ref[...] = jnp.concatenate(ids, axis=1).reshape(1, qb, _NSLOT)
    v9_ref[...] = jnp.max(v, axis=1, keepdims=True).reshape(1, qb, 1)


def _select_topk(dist, topk, qb, nb):
    Q, n_pad = dist.shape
    n_blocks = n_pad // nb
    vals, idxs, v9 = pl.pallas_call(
        functools.partial(_extract_block_kernel, n_pad, nb),
        grid=(Q // qb, n_blocks),
        in_specs=[pl.BlockSpec((qb, nb), lambda i, j: (i, j))],
        out_specs=[
            pl.BlockSpec((1, qb, _NSLOT), lambda i, j: (j, i, 0)),
            pl.BlockSpec((1, qb, _NSLOT), lambda i, j: (j, i, 0)),
            pl.BlockSpec((1, qb, 1), lambda i, j: (j, i, 0)),
        ],
        out_shape=[
            jax.ShapeDtypeStruct((n_blocks, Q, _NSLOT), jnp.float32),
            jax.ShapeDtypeStruct((n_blocks, Q, _NSLOT), jnp.int32),
            jax.ShapeDtypeStruct((n_blocks, Q, 1), jnp.float32),
        ],
    )(dist)
    cand_v = vals.transpose(1, 0, 2).reshape(Q, n_blocks * _NSLOT)
    cand_i = idxs.transpose(1, 0, 2).reshape(Q, n_blocks * _NSLOT)
    tv, tp = lax.top_k(cand_v, topk)
    fast_idx = jnp.take_along_axis(cand_i, tp, axis=1)
    kth = tv[:, topk - 1]
    v9max = jnp.max(v9[:, :, 0], axis=0)
    ok_fast = jnp.all(v9max < kth)
    return lax.cond(ok_fast,
                    lambda _: fast_idx,
                    lambda _: lax.top_k(dist, topk)[1],
                    operand=None)


def _split3(t):
    mask = jnp.uint32(0xFFFF0000)
    t0f = lax.bitcast_convert_type(
        lax.bitcast_convert_type(t, jnp.uint32) & mask, jnp.float32)
    r1 = t - t0f
    t1f = lax.bitcast_convert_type(
        lax.bitcast_convert_type(r1, jnp.uint32) & mask, jnp.float32)
    r2 = r1 - t1f
    return (t0f.astype(jnp.bfloat16), t1f.astype(jnp.bfloat16),
            r2.astype(jnp.bfloat16))


def _ivfpq_search(q, centroids, codebooks, db_assign, db_codes,
                  nprobe, topk, qb, nb):
    Q, D = q.shape
    C = centroids.shape[0]
    M, KK, DS = codebooks.shape
    N = db_assign.shape[0]

    qn = jnp.sum(q * q, axis=1, keepdims=True)
    cn = jnp.sum(centroids * centroids, axis=1)[None, :]
    coarse = qn + cn - 2.0 * (q @ centroids.T)
    tau = -lax.top_k(-coarse, nprobe)[0][:, -1:]

    qs = q.reshape(Q, M, DS)
    cs = centroids.reshape(C, M, DS)
    qdot = jnp.einsum('qmd,mkd->qmk', qs, codebooks)
    cdot = jnp.einsum('cmd,mkd->cmk', cs, codebooks)
    rnorm = jnp.sum(codebooks * codebooks, axis=2)

    table = jnp.concatenate([coarse, qdot.reshape(Q, M * KK)], axis=1)
    t0, t1, t2 = _split3(table)

    n_blocks = -(-N // nb)
    n_pad = n_blocks * nb
    pad = n_pad - N
    a_p = jnp.concatenate([db_assign, jnp.zeros((pad,), jnp.int32)])
    codes_p = jnp.concatenate([db_codes, jnp.zeros((pad, M), jnp.int32)], axis=0)
    offs = C + KK * jnp.arange(M, dtype=jnp.int32)
    keys = jnp.concatenate([a_p[None, :], (codes_p + offs[None, :]).T], axis=0)
    keys = keys.astype(jnp.int32)

    cdT = jnp.transpose(cdot.reshape(C, M * KK))
    c0, c1, c2 = _split3(cdT)
    rncol = rnorm.reshape(M * KK, 1)
    term3 = pl.pallas_call(
        functools.partial(_term_block_kernel, C, M, KK),
        grid=(n_blocks,),
        in_specs=[
            pl.BlockSpec((M * KK, C), lambda j: (0, 0)),
            pl.BlockSpec((M * KK, C), lambda j: (0, 0)),
            pl.BlockSpec((M * KK, C), lambda j: (0, 0)),
            pl.BlockSpec((M * KK, 1), lambda j: (0, 0)),
            pl.BlockSpec((1 + M, nb), lambda j: (0, j)),
        ],
        out_specs=pl.BlockSpec((1, 1, nb), lambda j: (j, 0, 0)),
        out_shape=jax.ShapeDtypeStruct((n_blocks, 1, nb), jnp.float32),
    )(c0, c1, c2, rncol, keys)
    if pad:
        tflat = term3.reshape(n_pad)
        tflat = jnp.where(jnp.arange(n_pad) >= N, jnp.inf, tflat)
        term3 = tflat.reshape(n_blocks, 1, nb)

    q_blocks = Q // qb
    dist = pl.pallas_call(
        functools.partial(_adc_block_kernel, C, M, KK),
        grid=(q_blocks, n_blocks),
        in_specs=[
            pl.BlockSpec((qb, C + M * KK), lambda i, j: (i, 0)),
            pl.BlockSpec((qb, C + M * KK), lambda i, j: (i, 0)),
            pl.BlockSpec((qb, C + M * KK), lambda i, j: (i, 0)),
            pl.BlockSpec((1 + M, nb), lambda i, j: (0, j)),
            pl.BlockSpec((1, 1, nb), lambda i, j: (j, 0, 0)),
            pl.BlockSpec((qb, 1), lambda i, j: (i, 0)),
        ],
        out_specs=pl.BlockSpec((qb, nb), lambda i, j: (i, j)),
        out_shape=jax.ShapeDtypeStruct((Q, n_pad), jnp.float32),
    )(t0, t1, t2, keys, term3, tau)

    return _select_topk(dist, topk, qb, nb)


def kernel(input_embedding, centroids, codebooks, db_assign, db_codes):
    return _ivfpq_search(input_embedding, centroids, codebooks,
                         db_assign, db_codes,
                         nprobe=_NPROBE, topk=_TOPK, qb=512, nb=1024)
